# baseline (speedup 1.0000x reference)
import functools

import jax
import jax.numpy as jnp
from jax import lax
from jax.experimental import pallas as pl
from jax.experimental.pallas import tpu as pltpu
from jax.experimental.pallas import tpu_sc as plsc

KM = 8
LANES = 16
SPATIAL = 1024
ROWS_PER_CHUNK = 16

_SORT8 = [(0, 1), (2, 3), (4, 5), (6, 7), (0, 2), (1, 3), (4, 6), (5, 7),
          (1, 2), (5, 6), (0, 4), (1, 5), (2, 6), (3, 7), (2, 4), (3, 5),
          (1, 2), (3, 4), (5, 6)]
_BITONIC8 = [(0, 4), (1, 5), (2, 6), (3, 7), (0, 2), (1, 3), (4, 6), (5, 7),
             (0, 1), (2, 3), (4, 5), (6, 7)]


def _apply_network(v, net):
    v = list(v)
    for i, j in net:
        hi = jnp.maximum(v[i], v[j])
        lo = jnp.minimum(v[i], v[j])
        v[i], v[j] = hi, lo
    return v


def _merge_top8(a, b):
    c = [jnp.maximum(a[i], b[7 - i]) for i in range(KM)]
    return _apply_network(c, _BITONIC8)


def _make_sc_topk(B, CG):
    mesh = plsc.VectorSubcoreMesh(core_axis_name="c", subcore_axis_name="s")
    info = plsc.get_sparse_core_info()
    nc = info.num_cores

    @functools.partial(
        pl.kernel,
        out_type=jax.ShapeDtypeStruct((B, CG, KM, LANES), jnp.float32),
        mesh=mesh,
        scratch_types=[
            pltpu.VMEM((SPATIAL, LANES), jnp.float32),
            pltpu.VMEM((KM, LANES), jnp.float32),
        ],
        compiler_params=pltpu.CompilerParams(use_tc_tiling_on_sc=False),
    )
    def topk_kernel(x_hbm, out_hbm, slab, stage):
        b = lax.axis_index("s") * nc + lax.axis_index("c")

        def per_group(cg, carry):
            pltpu.sync_copy(x_hbm.at[b, :, pl.ds(cg * LANES, LANES)], slab)

            def per_chunk(i, r):
                base = i * ROWS_PER_CHUNK
                rows = [slab[base + k, :] for k in range(ROWS_PER_CHUNK)]
                a = _apply_network(rows[:KM], _SORT8)
                bb = _apply_network(rows[KM:], _SORT8)
                c = _merge_top8(a, bb)
                return tuple(_merge_top8(list(r), c))

            neg_inf = jnp.full((LANES,), -jnp.inf, jnp.float32)
            r0 = (neg_inf,) * KM
            r = lax.fori_loop(0, SPATIAL // ROWS_PER_CHUNK, per_chunk, r0)
            for k in range(KM):
                stage[k, :] = r[k]
            pltpu.sync_copy(stage, out_hbm.at[b, cg])
            return carry

        lax.fori_loop(0, CG, per_group, 0)

    return topk_kernel


def kernel(x):
    B, H, W, C = x.shape
    CG = C // LANES
    xr = jnp.reshape(x, (B, H * W, C))
    out = _make_sc_topk(B, CG)(xr)
    out = jnp.transpose(out, (0, 1, 3, 2))
    return jnp.reshape(out, (B, KM * C))


# speedup vs baseline: 1.4559x; 1.4559x over previous
import functools

import jax
import jax.numpy as jnp
from jax import lax
from jax.experimental import pallas as pl
from jax.experimental.pallas import tpu as pltpu
from jax.experimental.pallas import tpu_sc as plsc

KM = 8
LANES = 16
SPATIAL = 1024
ROWS_PER_CHUNK = 16

_SORT8 = [(0, 1), (2, 3), (4, 5), (6, 7), (0, 2), (1, 3), (4, 6), (5, 7),
          (1, 2), (5, 6), (0, 4), (1, 5), (2, 6), (3, 7), (2, 4), (3, 5),
          (1, 2), (3, 4), (5, 6)]
_BITONIC8 = [(0, 4), (1, 5), (2, 6), (3, 7), (0, 2), (1, 3), (4, 6), (5, 7),
             (0, 1), (2, 3), (4, 5), (6, 7)]


def _apply_network(v, net):
    v = list(v)
    for i, j in net:
        hi = jnp.maximum(v[i], v[j])
        lo = jnp.minimum(v[i], v[j])
        v[i], v[j] = hi, lo
    return v


def _merge_top8(a, b):
    c = [jnp.maximum(a[i], b[7 - i]) for i in range(KM)]
    return _apply_network(c, _BITONIC8)


def _make_sc_topk(B, CG):
    mesh = plsc.VectorSubcoreMesh(core_axis_name="c", subcore_axis_name="s")
    info = plsc.get_sparse_core_info()
    nc = info.num_cores

    @functools.partial(
        pl.kernel,
        out_type=jax.ShapeDtypeStruct((B, CG, KM, LANES), jnp.float32),
        mesh=mesh,
        scratch_types=[
            pltpu.VMEM((SPATIAL // 8, 8, LANES), jnp.float32),
            pltpu.VMEM((KM, LANES), jnp.float32),
        ],
        compiler_params=pltpu.CompilerParams(use_tc_tiling_on_sc=False),
    )
    def topk_kernel(x_hbm, out_hbm, slab, stage):
        b = lax.axis_index("s") * nc + lax.axis_index("c")

        def per_group(cg, carry):
            ct = cg // 8
            lo = (cg % 8) * LANES
            pltpu.sync_copy(x_hbm.at[b, :, ct, :, pl.ds(lo, LANES)], slab)

            def per_chunk(i, r):
                st = i * 2
                rows = [slab[st + (k // 8), k % 8, :]
                        for k in range(ROWS_PER_CHUNK)]
                a = _apply_network(rows[:KM], _SORT8)
                bb = _apply_network(rows[KM:], _SORT8)
                c = _merge_top8(a, bb)
                return tuple(_merge_top8(list(r), c))

            neg_inf = jnp.full((LANES,), -jnp.inf, jnp.float32)
            r0 = (neg_inf,) * KM
            r = lax.fori_loop(0, SPATIAL // ROWS_PER_CHUNK, per_chunk, r0)
            for k in range(KM):
                stage[k, :] = r[k]
            pltpu.sync_copy(stage, out_hbm.at[b, cg])
            return carry

        lax.fori_loop(0, CG, per_group, 0)

    return topk_kernel


def kernel(x):
    B, H, W, C = x.shape
    CG = C // LANES
    xr = jnp.transpose(
        jnp.reshape(x, (B, H * W // 8, 8, C // 128, 128)), (0, 1, 3, 2, 4))
    out = _make_sc_topk(B, CG)(xr)
    out = jnp.transpose(out, (0, 1, 3, 2))
    return jnp.reshape(out, (B, KM * C))


# speedup vs baseline: 2.6597x; 1.8269x over previous
import functools

import jax
import jax.numpy as jnp
from jax import lax
from jax.experimental import pallas as pl
from jax.experimental.pallas import tpu as pltpu
from jax.experimental.pallas import tpu_sc as plsc

KM = 8
LANES = 16
SPATIAL = 1024
ST = SPATIAL // 8
SEC = 4
ST_SEC = ST // SEC
CT = 6
GROUPS = 8
ROWS_PER_CHUNK = 16

_SORT8 = [(0, 1), (2, 3), (4, 5), (6, 7), (0, 2), (1, 3), (4, 6), (5, 7),
          (1, 2), (5, 6), (0, 4), (1, 5), (2, 6), (3, 7), (2, 4), (3, 5),
          (1, 2), (3, 4), (5, 6)]
_BITONIC8 = [(0, 4), (1, 5), (2, 6), (3, 7), (0, 2), (1, 3), (4, 6), (5, 7),
             (0, 1), (2, 3), (4, 5), (6, 7)]


def _apply_network(v, net):
    v = list(v)
    for i, j in net:
        hi = jnp.maximum(v[i], v[j])
        lo = jnp.minimum(v[i], v[j])
        v[i], v[j] = hi, lo
    return v


def _merge_top8(a, b):
    c = [jnp.maximum(a[i], b[7 - i]) for i in range(KM)]
    return _apply_network(c, _BITONIC8)


def _make_sc_topk(B):
    mesh = plsc.VectorSubcoreMesh(core_axis_name="c", subcore_axis_name="s")
    info = plsc.get_sparse_core_info()
    nc = info.num_cores

    @functools.partial(
        pl.kernel,
        out_type=jax.ShapeDtypeStruct((B, CT, KM, 128), jnp.float32),
        mesh=mesh,
        scratch_types=[
            pltpu.VMEM((ST_SEC, 8, 128), jnp.float32),
            pltpu.VMEM((ST_SEC, 8, 128), jnp.float32),
            pltpu.VMEM((KM, 128), jnp.float32),
            pltpu.SemaphoreType.DMA,
            pltpu.SemaphoreType.DMA,
        ],
    )
    def topk_kernel(x_hbm, out_hbm, buf_a, buf_b, stage, sem_a, sem_b):
        b = lax.axis_index("s") * nc + lax.axis_index("c")
        bufs = (buf_a, buf_b)
        sems = (sem_a, sem_b)
        neg_inf = jnp.full((LANES,), -jnp.inf, jnp.float32)

        def start_dma(st0, ct, bi):
            pltpu.async_copy(x_hbm.at[b, pl.ds(st0, ST_SEC), ct, :, :],
                             bufs[bi], sems[bi])

        def wait_dma(bi):
            pltpu.make_async_copy(
                x_hbm.at[b, pl.ds(0, ST_SEC), 0, :, :],
                bufs[bi], sems[bi]).wait()

        start_dma(0, 0, 0)

        def per_ct(ct, carry):
            for sec in range(SEC):
                bi = sec % 2
                if sec < SEC - 1:
                    start_dma((sec + 1) * ST_SEC, ct, 1 - bi)
                else:
                    @pl.when(ct + 1 < CT)
                    def _():
                        start_dma(0, ct + 1, 1 - bi)
                wait_dma(bi)
                buf = bufs[bi]

                def per_group(g, carry_g):
                    lo = g * LANES
                    if sec == 0:
                        r0 = (neg_inf,) * KM
                    else:
                        r0 = tuple(stage[k, pl.ds(lo, LANES)]
                                   for k in range(KM))

                    def per_chunk(i, r):
                        st2 = i * 2
                        rows = [buf[st2 + (k // 8), k % 8, pl.ds(lo, LANES)]
                                for k in range(ROWS_PER_CHUNK)]
                        a = _apply_network(rows[:KM], _SORT8)
                        bb = _apply_network(rows[KM:], _SORT8)
                        c = _merge_top8(a, bb)
                        return tuple(_merge_top8(list(r), c))

                    n_chunks = (ST_SEC * 8) // ROWS_PER_CHUNK
                    r = lax.fori_loop(0, n_chunks, per_chunk, r0)
                    for k in range(KM):
                        stage[k, pl.ds(lo, LANES)] = r[k]
                    return carry_g

                lax.fori_loop(0, GROUPS, per_group, 0)
            pltpu.sync_copy(stage, out_hbm.at[b, ct])
            return carry

        lax.fori_loop(0, CT, per_ct, 0)

    return topk_kernel


def kernel(x):
    B, H, W, C = x.shape
    xr = jnp.transpose(
        jnp.reshape(x, (B, H * W // 8, 8, C // 128, 128)), (0, 1, 3, 2, 4))
    out = _make_sc_topk(B)(xr)
    out = jnp.transpose(out, (0, 1, 3, 2))
    return jnp.reshape(out, (B, KM * C))


# speedup vs baseline: 3.4238x; 1.2873x over previous
import functools

import jax
import jax.numpy as jnp
from jax import lax
from jax.experimental import pallas as pl
from jax.experimental.pallas import tpu as pltpu
from jax.experimental.pallas import tpu_sc as plsc

KM = 8
LANES = 16
SPATIAL = 1024
ST = SPATIAL // 8
SEC = 4
ST_SEC = ST // SEC
CT = 6
GROUPS = 8
ROWS_PER_CHUNK = 16
B_SC = 16

_SORT8 = [(0, 1), (2, 3), (4, 5), (6, 7), (0, 2), (1, 3), (4, 6), (5, 7),
          (1, 2), (5, 6), (0, 4), (1, 5), (2, 6), (3, 7), (2, 4), (3, 5),
          (1, 2), (3, 4), (5, 6)]
_BITONIC8 = [(0, 4), (1, 5), (2, 6), (3, 7), (0, 2), (1, 3), (4, 6), (5, 7),
             (0, 1), (2, 3), (4, 5), (6, 7)]


def _apply_network(v, net):
    v = list(v)
    for i, j in net:
        hi = jnp.maximum(v[i], v[j])
        lo = jnp.minimum(v[i], v[j])
        v[i], v[j] = hi, lo
    return v


def _merge_top8(a, b):
    c = [jnp.maximum(a[i], b[7 - i]) for i in range(KM)]
    return _apply_network(c, _BITONIC8)


def _fold_chunk16(rows, r):
    a = _apply_network(rows[:KM], _SORT8)
    b = _apply_network(rows[KM:], _SORT8)
    c = _merge_top8(a, b)
    return c if r is None else _merge_top8(list(r), c)


def _make_sc_topk():
    mesh = plsc.VectorSubcoreMesh(core_axis_name="c", subcore_axis_name="s")
    info = plsc.get_sparse_core_info()
    nc = info.num_cores
    cts_per_worker = CT * B_SC // 32

    @functools.partial(
        pl.kernel,
        out_type=jax.ShapeDtypeStruct((B_SC, CT, KM, 128), jnp.float32),
        mesh=mesh,
        scratch_types=[
            pltpu.VMEM((ST_SEC, 8, 128), jnp.float32),
            pltpu.VMEM((ST_SEC, 8, 128), jnp.float32),
            pltpu.VMEM((KM, 128), jnp.float32),
            pltpu.SemaphoreType.DMA,
            pltpu.SemaphoreType.DMA,
        ],
    )
    def topk_kernel(x_hbm, out_hbm, buf_a, buf_b, stage, sem_a, sem_b):
        w = lax.axis_index("s") * nc + lax.axis_index("c")
        b = w // 2
        ct_base = (w % 2) * cts_per_worker
        bufs = (buf_a, buf_b)
        sems = (sem_a, sem_b)
        neg_inf = jnp.full((LANES,), -jnp.inf, jnp.float32)

        def start_dma(st0, ct, bi):
            pltpu.async_copy(x_hbm.at[b, pl.ds(st0, ST_SEC), ct, :, :],
                             bufs[bi], sems[bi])

        def wait_dma(bi):
            pltpu.make_async_copy(
                x_hbm.at[b, pl.ds(0, ST_SEC), 0, :, :],
                bufs[bi], sems[bi]).wait()

        start_dma(0, ct_base, 0)

        def per_ct(i, carry):
            ct = ct_base + i
            for sec in range(SEC):
                bi = sec % 2
                if sec < SEC - 1:
                    start_dma((sec + 1) * ST_SEC, ct, 1 - bi)
                else:
                    @pl.when(i + 1 < cts_per_worker)
                    def _():
                        start_dma(0, ct + 1, 1 - bi)
                wait_dma(bi)
                buf = bufs[bi]

                def per_group(g, carry_g):
                    lo = g * LANES
                    if sec == 0:
                        r0 = (neg_inf,) * KM
                    else:
                        r0 = tuple(stage[k, pl.ds(lo, LANES)]
                                   for k in range(KM))

                    def per_chunk(ic, r):
                        st2 = ic * 2
                        rows = [buf[st2 + (k // 8), k % 8, pl.ds(lo, LANES)]
                                for k in range(ROWS_PER_CHUNK)]
                        return tuple(_fold_chunk16(rows, r))

                    n_chunks = (ST_SEC * 8) // ROWS_PER_CHUNK
                    r = lax.fori_loop(0, n_chunks, per_chunk, r0)
                    for k in range(KM):
                        stage[k, pl.ds(lo, LANES)] = r[k]
                    return carry_g

                lax.fori_loop(0, GROUPS, per_group, 0)
            pltpu.sync_copy(stage, out_hbm.at[b, ct])
            return carry

        lax.fori_loop(0, cts_per_worker, per_ct, 0)

    return topk_kernel


def _tc_body(x_ref, o_ref):
    r = None
    for ci in range(SPATIAL // (ROWS_PER_CHUNK * 8)):
        rows = [x_ref[0, pl.ds((ci * ROWS_PER_CHUNK + k) * 8, 8), :]
                for k in range(ROWS_PER_CHUNK)]
        r = _fold_chunk16(rows, r)
    r = list(r)
    for shift in (4, 2, 1):
        rolled = [pltpu.roll(r[7 - i], shift, 0) for i in range(KM)]
        c = [jnp.maximum(r[i], rolled[i]) for i in range(KM)]
        r = _apply_network(c, _BITONIC8)
    sub = lax.broadcasted_iota(jnp.int32, (KM, 128), 0)
    acc = r[0]
    for k in range(1, KM):
        acc = jnp.where(sub == k, r[k], acc)
    o_ref[0, 0] = acc


def _make_tc_topk(b_tc, b_off):
    return pl.pallas_call(
        _tc_body,
        grid=(b_tc, CT),
        in_specs=[pl.BlockSpec((1, SPATIAL, 128),
                               lambda b, ct: (b + b_off, 0, ct))],
        out_specs=pl.BlockSpec((1, 1, KM, 128),
                               lambda b, ct: (b, ct, 0, 0)),
        out_shape=jax.ShapeDtypeStruct((b_tc, CT, KM, 128), jnp.float32),
    )


def kernel(x):
    B, H, W, C = x.shape
    xr = jnp.transpose(
        jnp.reshape(x, (B, H * W // 8, 8, C // 128, 128)), (0, 1, 3, 2, 4))
    out_sc = _make_sc_topk()(xr)
    x3 = jnp.reshape(x, (B, H * W, C))
    out_tc = _make_tc_topk(B - B_SC, B_SC)(x3)
    out = jnp.concatenate([out_sc, out_tc], axis=0)
    out = jnp.transpose(out, (0, 1, 3, 2))
    return jnp.reshape(out, (B, KM * C))


# speedup vs baseline: 4.1936x; 1.2248x over previous
import functools

import jax
import jax.numpy as jnp
from jax import lax
from jax.experimental import pallas as pl
from jax.experimental.pallas import tpu as pltpu
from jax.experimental.pallas import tpu_sc as plsc

KM = 8
LANES = 16
SPATIAL = 1024
ST = SPATIAL // 8
SEC = 4
ST_SEC = ST // SEC
CT = 6
GROUPS = 8
ROWS_PER_CHUNK = 16
B_SC = 16

_SORT8 = [(0, 1), (2, 3), (4, 5), (6, 7), (0, 2), (1, 3), (4, 6), (5, 7),
          (1, 2), (5, 6), (0, 4), (1, 5), (2, 6), (3, 7), (2, 4), (3, 5),
          (1, 2), (3, 4), (5, 6)]
_BITONIC8 = [(0, 4), (1, 5), (2, 6), (3, 7), (0, 2), (1, 3), (4, 6), (5, 7),
             (0, 1), (2, 3), (4, 5), (6, 7)]


def _apply_network(v, net):
    v = list(v)
    for i, j in net:
        hi = jnp.maximum(v[i], v[j])
        lo = jnp.minimum(v[i], v[j])
        v[i], v[j] = hi, lo
    return v


def _merge_top8(a, b):
    c = [jnp.maximum(a[i], b[7 - i]) for i in range(KM)]
    return _apply_network(c, _BITONIC8)


def _fold_chunk16(rows, r):
    a = _apply_network(rows[:KM], _SORT8)
    b = _apply_network(rows[KM:], _SORT8)
    c = _merge_top8(a, b)
    return c if r is None else _merge_top8(list(r), c)


def _make_sc_topk():
    mesh = plsc.VectorSubcoreMesh(core_axis_name="c", subcore_axis_name="s")
    info = plsc.get_sparse_core_info()
    nc = info.num_cores
    cts_per_worker = CT * B_SC // 32

    @functools.partial(
        pl.kernel,
        out_type=jax.ShapeDtypeStruct((B_SC, CT, KM, 128), jnp.float32),
        mesh=mesh,
        scratch_types=[
            pltpu.VMEM((ST_SEC, 8, 128), jnp.float32),
            pltpu.VMEM((ST_SEC, 8, 128), jnp.float32),
            pltpu.VMEM((KM, 128), jnp.float32),
            pltpu.SemaphoreType.DMA,
            pltpu.SemaphoreType.DMA,
        ],
    )
    def topk_kernel(x_hbm, out_hbm, buf_a, buf_b, stage, sem_a, sem_b):
        w = lax.axis_index("s") * nc + lax.axis_index("c")
        b = w // 2
        ct_base = (w % 2) * cts_per_worker
        bufs = (buf_a, buf_b)
        sems = (sem_a, sem_b)
        neg_inf = jnp.full((LANES,), -jnp.inf, jnp.float32)

        def start_dma(st0, ct, bi):
            pltpu.async_copy(x_hbm.at[b, pl.ds(st0, ST_SEC), ct, :, :],
                             bufs[bi], sems[bi])

        def wait_dma(bi):
            pltpu.make_async_copy(
                x_hbm.at[b, pl.ds(0, ST_SEC), 0, :, :],
                bufs[bi], sems[bi]).wait()

        start_dma(0, ct_base, 0)

        def per_ct(i, carry):
            ct = ct_base + i
            for sec in range(SEC):
                bi = sec % 2
                if sec < SEC - 1:
                    start_dma((sec + 1) * ST_SEC, ct, 1 - bi)
                else:
                    @pl.when(i + 1 < cts_per_worker)
                    def _():
                        start_dma(0, ct + 1, 1 - bi)
                wait_dma(bi)
                buf = bufs[bi]

                def per_group(g, carry_g):
                    lo = g * LANES
                    if sec == 0:
                        r0 = (neg_inf,) * KM
                    else:
                        r0 = tuple(stage[k, pl.ds(lo, LANES)]
                                   for k in range(KM))

                    def per_chunk(ic, r):
                        st2 = ic * 2
                        rows = [buf[st2 + (k // 8), k % 8, pl.ds(lo, LANES)]
                                for k in range(ROWS_PER_CHUNK)]
                        return tuple(_fold_chunk16(rows, r))

                    n_chunks = (ST_SEC * 8) // ROWS_PER_CHUNK
                    r = lax.fori_loop(0, n_chunks, per_chunk, r0)
                    for k in range(KM):
                        stage[k, pl.ds(lo, LANES)] = r[k]
                    return carry_g

                lax.fori_loop(0, GROUPS, per_group, 0)
            pltpu.sync_copy(stage, out_hbm.at[b, ct])
            return carry

        lax.fori_loop(0, cts_per_worker, per_ct, 0)

    return topk_kernel


def _tc_column_top8(x_ref):
    r = None
    for ci in range(SPATIAL // (ROWS_PER_CHUNK * 8)):
        rows = [x_ref[0, pl.ds((ci * ROWS_PER_CHUNK + k) * 8, 8), :]
                for k in range(ROWS_PER_CHUNK)]
        r = _fold_chunk16(rows, r)
    r = list(r)
    for shift in (4, 2, 1):
        rolled = [pltpu.roll(r[7 - i], shift, 0) for i in range(KM)]
        c = [jnp.maximum(r[i], rolled[i]) for i in range(KM)]
        r = _apply_network(c, _BITONIC8)
    sub = lax.broadcasted_iota(jnp.int32, (KM, 128), 0)
    acc = r[0]
    for k in range(1, KM):
        acc = jnp.where(sub == k, r[k], acc)
    return acc


def _tc_body(xa_ref, xb_ref, o_ref):
    o_ref[0, 0] = _tc_column_top8(xa_ref)
    o_ref[0, 1] = _tc_column_top8(xb_ref)


def _make_tc_topk(b_tc, b_off):
    return pl.pallas_call(
        _tc_body,
        grid=(b_tc, CT // 2),
        in_specs=[pl.BlockSpec((1, SPATIAL, 128),
                               lambda b, i: (b + b_off, 0, 2 * i)),
                  pl.BlockSpec((1, SPATIAL, 128),
                               lambda b, i: (b + b_off, 0, 2 * i + 1))],
        out_specs=pl.BlockSpec((1, 2, KM, 128),
                               lambda b, i: (b, i, 0, 0)),
        out_shape=jax.ShapeDtypeStruct((b_tc, CT, KM, 128), jnp.float32),
    )


def kernel(x):
    B, H, W, C = x.shape
    xr = jnp.transpose(
        jnp.reshape(x, (B, H * W // 8, 8, C // 128, 128)), (0, 1, 3, 2, 4))
    out_sc = _make_sc_topk()(xr)
    x3 = jnp.reshape(x, (B, H * W, C))
    out_tc = _make_tc_topk(B - B_SC, B_SC)(x3, x3)
    out = jnp.concatenate([out_sc, out_tc], axis=0)
    out = jnp.transpose(out, (0, 1, 3, 2))
    return jnp.reshape(out, (B, KM * C))
